# baseline (speedup 1.0000x reference)
import functools

import jax
import jax.numpy as jnp
from jax import lax
from jax.experimental import pallas as pl
from jax.experimental.pallas import tpu as pltpu
from jax.experimental.pallas import tpu_sc as plsc

N = 10000
E = 320000
D = 128
NCLS = 40

NC = 2
NS = 16
NW = NC * NS
EPW = E // NW
CHUNK = 64
WIN = 8
NGRP = 20
NCH = WIN * NGRP
EPAD = NCH * CHUNK
N_P = 10240
RPT = N_P // NS

_MESH = plsc.VectorSubcoreMesh(
    core_axis_name="c", subcore_axis_name="s", num_cores=NC, num_subcores=NS
)


@functools.partial(
    pl.kernel,
    out_type=jax.ShapeDtypeStruct((NC, NS, 2, RPT), jnp.float32),
    mesh=_MESH,
    scratch_types=[
        pltpu.VMEM((2, WIN, CHUNK), jnp.int32),
        pltpu.VMEM((2, WIN, CHUNK), jnp.int32),
        pltpu.VMEM((CHUNK,), jnp.float32),
        pltpu.VMEM((RPT,), jnp.float32),
        pltpu.VMEM_SHARED((N_P,), jnp.float32),
        pltpu.VMEM_SHARED((N_P,), jnp.float32),
        [pltpu.SemaphoreType.DMA] * 2,
        [pltpu.SemaphoreType.DMA] * 2,
        pltpu.SemaphoreType.DMA,
    ],
)
def _deg_kernel(edges_hbm, degp_hbm,
                idxs_w, idxd_w, ones_v, stage_v, degs_sp, degd_sp,
                isems, isemd, dsem):
    c = lax.axis_index("c")
    s = lax.axis_index("s")
    w = c * NS + s
    off = pl.multiple_of(s * RPT, 8)
    for k in range(RPT // 16):
        stage_v[pl.ds(16 * k, 16)] = jnp.zeros((16,), jnp.float32)
    pltpu.sync_copy(stage_v, degs_sp.at[pl.ds(off, RPT)])
    pltpu.sync_copy(stage_v, degd_sp.at[pl.ds(off, RPT)])
    for k in range(CHUNK // 16):
        ones_v[pl.ds(16 * k, 16)] = jnp.ones((16,), jnp.float32)
    plsc.subcore_barrier()

    pltpu.async_copy(edges_hbm.at[0, w, 0], idxs_w.at[0], isems[0])
    pltpu.async_copy(edges_hbm.at[1, w, 0], idxd_w.at[0], isemd[0])

    def outer(g2, carry):
        for gg in range(2):
            g = g2 * 2 + gg
            sl, sl2 = gg, 1 - gg
            pltpu.make_async_copy(
                edges_hbm.at[0, w, 0], idxs_w.at[sl], isems[sl]
            ).wait()
            pltpu.make_async_copy(
                edges_hbm.at[1, w, 0], idxd_w.at[sl], isemd[sl]
            ).wait()

            for k in range(WIN):
                pltpu.async_copy(ones_v, degs_sp.at[idxs_w.at[sl, k]], dsem,
                                 add=True)
                pltpu.async_copy(ones_v, degd_sp.at[idxd_w.at[sl, k]], dsem,
                                 add=True)

            @pl.when(g + 1 < NGRP)
            def _():
                pltpu.async_copy(edges_hbm.at[0, w, g + 1], idxs_w.at[sl2],
                                 isems[sl2])
                pltpu.async_copy(edges_hbm.at[1, w, g + 1], idxd_w.at[sl2],
                                 isemd[sl2])

            for k in range(WIN):
                pltpu.make_async_copy(
                    ones_v, degs_sp.at[idxs_w.at[sl, k]], dsem
                ).wait()
                pltpu.make_async_copy(
                    ones_v, degd_sp.at[idxd_w.at[sl, k]], dsem
                ).wait()
        return carry

    lax.fori_loop(0, NGRP // 2, outer, 0)
    plsc.subcore_barrier()
    pltpu.sync_copy(degs_sp.at[pl.ds(off, RPT)], stage_v)
    pltpu.sync_copy(stage_v, degp_hbm.at[c, s, 0])
    pltpu.sync_copy(degd_sp.at[pl.ds(off, RPT)], stage_v)
    pltpu.sync_copy(stage_v, degp_hbm.at[c, s, 1])


@functools.partial(
    pl.kernel,
    out_type=jax.ShapeDtypeStruct((NC, N_P, D), jnp.float32),
    mesh=_MESH,
    scratch_types=[
        pltpu.VMEM((2, WIN, CHUNK), jnp.int32),
        pltpu.VMEM((2, WIN, CHUNK), jnp.int32),
        [pltpu.VMEM((CHUNK, D), jnp.float32)] * 4,
        pltpu.VMEM_SHARED((N_P, D), jnp.float32),
        [pltpu.SemaphoreType.DMA] * 4,
        [pltpu.SemaphoreType.DMA] * 4,
        [pltpu.SemaphoreType.DMA] * 2,
        [pltpu.SemaphoreType.DMA] * 2,
    ],
)
def _agg_kernel(xh_hbm, edges_hbm, part_hbm,
                idxg_w, idxw_w, rows, acc_sp, gsem, ssem, isemg, isemw):
    c = lax.axis_index("c")
    s = lax.axis_index("s")
    w = c * NS + s
    off = pl.multiple_of(s * RPT, 8)
    def zrow(r, carry):
        for k in range(D // 16):
            rows[0][r, pl.ds(16 * k, 16)] = jnp.zeros((16,), jnp.float32)
        return carry

    lax.fori_loop(0, CHUNK, zrow, 0)
    for i in range(RPT // CHUNK):
        pltpu.async_copy(rows[0], acc_sp.at[pl.ds(off + i * CHUNK, CHUNK)],
                         gsem[0])
    for i in range(RPT // CHUNK):
        pltpu.make_async_copy(
            rows[0], acc_sp.at[pl.ds(off + i * CHUNK, CHUNK)], gsem[0]
        ).wait()
    plsc.subcore_barrier()

    pltpu.async_copy(edges_hbm.at[0, w, 0], idxg_w.at[0], isemg[0])
    pltpu.async_copy(edges_hbm.at[1, w, 0], idxw_w.at[0], isemw[0])
    pltpu.make_async_copy(edges_hbm.at[0, w, 0], idxg_w.at[0], isemg[0]).wait()
    pltpu.make_async_copy(edges_hbm.at[1, w, 0], idxw_w.at[0], isemw[0]).wait()
    pltpu.async_copy(xh_hbm.at[idxg_w.at[0, 0]], rows[0], gsem[0])
    pltpu.async_copy(xh_hbm.at[idxg_w.at[0, 1]], rows[1], gsem[1])

    def outer(g2, carry):
        for gg in range(2):
            g = g2 * 2 + gg
            sl, sl2 = gg, 1 - gg
            for k in range(WIN):
                j = g * WIN + k
                b = k % 4
                bs = (k + 2) % 4
                pltpu.make_async_copy(
                    xh_hbm.at[idxg_w.at[sl, k]], rows[b], gsem[b]
                ).wait()
                pltpu.async_copy(rows[b], acc_sp.at[idxw_w.at[sl, k]],
                                 ssem[b], add=True)

                if k < 2:
                    @pl.when(j >= 2)
                    def _():
                        pltpu.make_async_copy(
                            rows[bs], acc_sp.at[idxw_w.at[sl2, WIN - 2 + k]],
                            ssem[bs],
                        ).wait()
                else:
                    pltpu.make_async_copy(
                        rows[bs], acc_sp.at[idxw_w.at[sl, k - 2]], ssem[bs]
                    ).wait()

                if k == 2:
                    @pl.when(g + 1 < NGRP)
                    def _():
                        pltpu.async_copy(edges_hbm.at[0, w, g + 1],
                                         idxg_w.at[sl2], isemg[sl2])
                        pltpu.async_copy(edges_hbm.at[1, w, g + 1],
                                         idxw_w.at[sl2], isemw[sl2])

                if k < WIN - 2:
                    pltpu.async_copy(xh_hbm.at[idxg_w.at[sl, k + 2]],
                                     rows[bs], gsem[bs])
                elif k == WIN - 2:
                    @pl.when(j + 2 < NCH)
                    def _():
                        pltpu.make_async_copy(
                            edges_hbm.at[0, w, 0], idxg_w.at[sl2], isemg[sl2]
                        ).wait()
                        pltpu.make_async_copy(
                            edges_hbm.at[1, w, 0], idxw_w.at[sl2], isemw[sl2]
                        ).wait()
                        pltpu.async_copy(xh_hbm.at[idxg_w.at[sl2, 0]],
                                         rows[bs], gsem[bs])
                else:
                    @pl.when(j + 2 < NCH)
                    def _():
                        pltpu.async_copy(xh_hbm.at[idxg_w.at[sl2, 1]],
                                         rows[bs], gsem[bs])
        return carry

    lax.fori_loop(0, NGRP // 2, outer, 0)
    for t in range(2):
        jj = NCH - 2 + t
        pltpu.make_async_copy(
            rows[jj % 4], acc_sp.at[idxw_w.at[(NGRP - 1) % 2, WIN - 2 + t]],
            ssem[jj % 4],
        ).wait()
    plsc.subcore_barrier()
    pltpu.sync_copy(acc_sp.at[pl.ds(off, RPT)], part_hbm.at[c, pl.ds(off, RPT)])


def _norms_body(degp_ref, out_ref):
    d = degp_ref[0] + degp_ref[1]
    out_ref[...] = lax.rsqrt(jnp.maximum(d, 1.0))


_norms_tc = pl.pallas_call(
    _norms_body,
    out_shape=jax.ShapeDtypeStruct((2, N_P // 128, 128), jnp.float32),
)


def _xw_body(x_ref, w_ref, o_ref):
    o_ref[pl.ds(0, N)] = jnp.dot(
        x_ref[...], w_ref[...], preferred_element_type=jnp.float32
    )
    o_ref[pl.ds(N, N_P - N)] = jnp.zeros((N_P - N, D), jnp.float32)


_xw_tc = pl.pallas_call(
    _xw_body,
    out_shape=jax.ShapeDtypeStruct((N_P, D), jnp.float32),
)


def _scale_body(xw_ref, ns_ref, o_ref):
    o_ref[...] = xw_ref[...] * ns_ref[...]


_scale_tc = pl.pallas_call(
    _scale_body,
    out_shape=jax.ShapeDtypeStruct((N_P, D), jnp.float32),
)


def _mid_body(p_ref, nd_ref, b_ref, w_ref, ns_ref, o_ref):
    agg = (p_ref[0, :N, :] + p_ref[1, :N, :]) * nd_ref[...] + b_ref[...]
    h = jnp.maximum(agg, 0.0)
    o_ref[pl.ds(0, N)] = (
        jnp.dot(h, w_ref[...], preferred_element_type=jnp.float32) * ns_ref[...]
    )
    o_ref[pl.ds(N, N_P - N)] = jnp.zeros((N_P - N, D), jnp.float32)


_mid_tc = pl.pallas_call(
    _mid_body,
    out_shape=jax.ShapeDtypeStruct((N_P, D), jnp.float32),
)


def _final_body(p_ref, nd_ref, b_ref, wfc_ref, bfc_ref, o_ref):
    agg = (p_ref[0, :N, :] + p_ref[1, :N, :]) * nd_ref[...] + b_ref[...]
    o_ref[...] = (
        jnp.dot(agg, wfc_ref[...], preferred_element_type=jnp.float32)
        + bfc_ref[...]
    )


_final_tc = pl.pallas_call(
    _final_body,
    out_shape=jax.ShapeDtypeStruct((N, NCLS), jnp.float32),
)


def kernel(inputs, edge_index, W1, b1, W2, b2, Wfc, bfc):
    pad = EPAD - EPW
    ei = edge_index.reshape(2, NW, EPW)
    pad_rows = jnp.broadcast_to(N + jnp.arange(pad, dtype=jnp.int32),
                                (2, NW, pad))
    edges = jnp.concatenate([ei, pad_rows], axis=2).reshape(
        2, NW, NGRP, WIN, CHUNK)

    xw1 = _xw_tc(inputs, W1)
    degp = _deg_kernel(edges)
    degp = degp.transpose(0, 2, 1, 3).reshape(NC, 2, N_P // 128, 128)
    norms = _norms_tc(degp)
    ns_colp = norms[0].reshape(N_P, 1)
    ns_col = ns_colp[:N]
    nd_col = norms[1].reshape(N_P, 1)[:N]

    xh1 = _scale_tc(xw1, ns_colp)
    p1 = _agg_kernel(xh1, edges)
    xh2 = _mid_tc(p1, nd_col, b1.reshape(1, D), W2, ns_col)
    p2 = _agg_kernel(xh2, edges)
    out = _final_tc(p2, nd_col, b2.reshape(1, D), Wfc, bfc.reshape(1, NCLS))
    return out


# speedup vs baseline: 1.1449x; 1.1449x over previous
import functools

import jax
import jax.numpy as jnp
from jax import lax
from jax.experimental import pallas as pl
from jax.experimental.pallas import tpu as pltpu
from jax.experimental.pallas import tpu_sc as plsc

N = 10000
E = 320000
D = 128
NCLS = 40

NC = 2
NS = 16
NW = NC * NS
EPW = E // NW
CHUNK = 96
WIN = 6
NGRP = 18
NCH = WIN * NGRP
EPAD = NCH * CHUNK
N_P = 10240
RPT = N_P // NS

_MESH = plsc.VectorSubcoreMesh(
    core_axis_name="c", subcore_axis_name="s", num_cores=NC, num_subcores=NS
)


@functools.partial(
    pl.kernel,
    out_type=jax.ShapeDtypeStruct((NC, NS, 2, RPT), jnp.float32),
    mesh=_MESH,
    scratch_types=[
        pltpu.VMEM((2, WIN, CHUNK), jnp.int32),
        pltpu.VMEM((2, WIN, CHUNK), jnp.int32),
        pltpu.VMEM((CHUNK,), jnp.float32),
        pltpu.VMEM((RPT,), jnp.float32),
        pltpu.VMEM_SHARED((N_P,), jnp.float32),
        pltpu.VMEM_SHARED((N_P,), jnp.float32),
        [pltpu.SemaphoreType.DMA] * 2,
        [pltpu.SemaphoreType.DMA] * 2,
        pltpu.SemaphoreType.DMA,
    ],
)
def _deg_kernel(edges_hbm, degp_hbm,
                idxs_w, idxd_w, ones_v, stage_v, degs_sp, degd_sp,
                isems, isemd, dsem):
    c = lax.axis_index("c")
    s = lax.axis_index("s")
    w = c * NS + s
    off = pl.multiple_of(s * RPT, 8)
    for k in range(RPT // 16):
        stage_v[pl.ds(16 * k, 16)] = jnp.zeros((16,), jnp.float32)
    pltpu.sync_copy(stage_v, degs_sp.at[pl.ds(off, RPT)])
    pltpu.sync_copy(stage_v, degd_sp.at[pl.ds(off, RPT)])
    for k in range(CHUNK // 16):
        ones_v[pl.ds(16 * k, 16)] = jnp.ones((16,), jnp.float32)
    plsc.subcore_barrier()

    pltpu.async_copy(edges_hbm.at[0, w, 0], idxs_w.at[0], isems[0])
    pltpu.async_copy(edges_hbm.at[1, w, 0], idxd_w.at[0], isemd[0])

    def outer(g2, carry):
        for gg in range(2):
            g = g2 * 2 + gg
            sl, sl2 = gg, 1 - gg
            pltpu.make_async_copy(
                edges_hbm.at[0, w, 0], idxs_w.at[sl], isems[sl]
            ).wait()
            pltpu.make_async_copy(
                edges_hbm.at[1, w, 0], idxd_w.at[sl], isemd[sl]
            ).wait()

            for k in range(WIN):
                pltpu.async_copy(ones_v, degs_sp.at[idxs_w.at[sl, k]], dsem,
                                 add=True)
                pltpu.async_copy(ones_v, degd_sp.at[idxd_w.at[sl, k]], dsem,
                                 add=True)

            @pl.when(g + 1 < NGRP)
            def _():
                pltpu.async_copy(edges_hbm.at[0, w, g + 1], idxs_w.at[sl2],
                                 isems[sl2])
                pltpu.async_copy(edges_hbm.at[1, w, g + 1], idxd_w.at[sl2],
                                 isemd[sl2])

            for k in range(WIN):
                pltpu.make_async_copy(
                    ones_v, degs_sp.at[idxs_w.at[sl, k]], dsem
                ).wait()
                pltpu.make_async_copy(
                    ones_v, degd_sp.at[idxd_w.at[sl, k]], dsem
                ).wait()
        return carry

    lax.fori_loop(0, NGRP // 2, outer, 0)
    plsc.subcore_barrier()
    pltpu.sync_copy(degs_sp.at[pl.ds(off, RPT)], stage_v)
    pltpu.sync_copy(stage_v, degp_hbm.at[c, s, 0])
    pltpu.sync_copy(degd_sp.at[pl.ds(off, RPT)], stage_v)
    pltpu.sync_copy(stage_v, degp_hbm.at[c, s, 1])


@functools.partial(
    pl.kernel,
    out_type=jax.ShapeDtypeStruct((NC, N_P, D), jnp.float32),
    mesh=_MESH,
    scratch_types=[
        pltpu.VMEM((2, WIN, CHUNK), jnp.int32),
        pltpu.VMEM((2, WIN, CHUNK), jnp.int32),
        [pltpu.VMEM((CHUNK, D), jnp.float32)] * 3,
        pltpu.VMEM_SHARED((N_P, D), jnp.float32),
        [pltpu.SemaphoreType.DMA] * 3,
        [pltpu.SemaphoreType.DMA] * 3,
        [pltpu.SemaphoreType.DMA] * 2,
        [pltpu.SemaphoreType.DMA] * 2,
    ],
)
def _agg_kernel(xh_hbm, edges_hbm, part_hbm,
                idxg_w, idxw_w, rows, acc_sp, gsem, ssem, isemg, isemw):
    c = lax.axis_index("c")
    s = lax.axis_index("s")
    w = c * NS + s
    off = pl.multiple_of(s * RPT, 8)
    def zrow(r, carry):
        for k in range(D // 16):
            rows[0][r, pl.ds(16 * k, 16)] = jnp.zeros((16,), jnp.float32)
        return carry

    lax.fori_loop(0, CHUNK, zrow, 0)
    for i in range(RPT // 80):
        pltpu.async_copy(rows[0].at[pl.ds(0, 80)],
                         acc_sp.at[pl.ds(off + i * 80, 80)], gsem[0])
    for i in range(RPT // 80):
        pltpu.make_async_copy(
            rows[0].at[pl.ds(0, 80)], acc_sp.at[pl.ds(off + i * 80, 80)],
            gsem[0],
        ).wait()
    plsc.subcore_barrier()

    pltpu.async_copy(edges_hbm.at[0, w, 0], idxg_w.at[0], isemg[0])
    pltpu.async_copy(edges_hbm.at[1, w, 0], idxw_w.at[0], isemw[0])
    pltpu.make_async_copy(edges_hbm.at[0, w, 0], idxg_w.at[0], isemg[0]).wait()
    pltpu.make_async_copy(edges_hbm.at[1, w, 0], idxw_w.at[0], isemw[0]).wait()
    pltpu.async_copy(xh_hbm.at[idxg_w.at[0, 0]], rows[0], gsem[0])
    pltpu.async_copy(xh_hbm.at[idxg_w.at[0, 1]], rows[1], gsem[1])

    def outer(g2, carry):
        for gg in range(2):
            g = g2 * 2 + gg
            sl, sl2 = gg, 1 - gg
            for k in range(WIN):
                j = g * WIN + k
                b = k % 3
                bs = (k + 2) % 3
                pltpu.make_async_copy(
                    xh_hbm.at[idxg_w.at[sl, k]], rows[b], gsem[b]
                ).wait()
                pltpu.async_copy(rows[b], acc_sp.at[idxw_w.at[sl, k]],
                                 ssem[b], add=True)

                if k == 0:
                    @pl.when(j >= 1)
                    def _():
                        pltpu.make_async_copy(
                            rows[bs], acc_sp.at[idxw_w.at[sl2, WIN - 1]],
                            ssem[bs],
                        ).wait()
                else:
                    pltpu.make_async_copy(
                        rows[bs], acc_sp.at[idxw_w.at[sl, k - 1]], ssem[bs]
                    ).wait()

                if k == 1:
                    @pl.when(g + 1 < NGRP)
                    def _():
                        pltpu.async_copy(edges_hbm.at[0, w, g + 1],
                                         idxg_w.at[sl2], isemg[sl2])
                        pltpu.async_copy(edges_hbm.at[1, w, g + 1],
                                         idxw_w.at[sl2], isemw[sl2])

                if k < WIN - 2:
                    pltpu.async_copy(xh_hbm.at[idxg_w.at[sl, k + 2]],
                                     rows[bs], gsem[bs])
                elif k == WIN - 2:
                    @pl.when(j + 2 < NCH)
                    def _():
                        pltpu.make_async_copy(
                            edges_hbm.at[0, w, 0], idxg_w.at[sl2], isemg[sl2]
                        ).wait()
                        pltpu.make_async_copy(
                            edges_hbm.at[1, w, 0], idxw_w.at[sl2], isemw[sl2]
                        ).wait()
                        pltpu.async_copy(xh_hbm.at[idxg_w.at[sl2, 0]],
                                         rows[bs], gsem[bs])
                else:
                    @pl.when(j + 2 < NCH)
                    def _():
                        pltpu.async_copy(xh_hbm.at[idxg_w.at[sl2, 1]],
                                         rows[bs], gsem[bs])
        return carry

    lax.fori_loop(0, NGRP // 2, outer, 0)
    pltpu.make_async_copy(
        rows[(NCH - 1) % 3], acc_sp.at[idxw_w.at[(NGRP - 1) % 2, WIN - 1]],
        ssem[(NCH - 1) % 3],
    ).wait()
    plsc.subcore_barrier()
    pltpu.sync_copy(acc_sp.at[pl.ds(off, RPT)], part_hbm.at[c, pl.ds(off, RPT)])


def _norms_body(degp_ref, out_ref):
    d = degp_ref[0] + degp_ref[1]
    out_ref[...] = lax.rsqrt(jnp.maximum(d, 1.0))


_norms_tc = pl.pallas_call(
    _norms_body,
    out_shape=jax.ShapeDtypeStruct((2, N_P // 128, 128), jnp.float32),
)


def _xw_scale_body(x_ref, w_ref, ns_ref, o_ref):
    o_ref[pl.ds(0, N)] = (
        jnp.dot(x_ref[...], w_ref[...], preferred_element_type=jnp.float32)
        * ns_ref[...]
    )
    o_ref[pl.ds(N, N_P - N)] = jnp.zeros((N_P - N, D), jnp.float32)


_xw_scale_tc = pl.pallas_call(
    _xw_scale_body,
    out_shape=jax.ShapeDtypeStruct((N_P, D), jnp.float32),
)


def _mid_body(p_ref, nd_ref, b_ref, w_ref, ns_ref, o_ref):
    agg = (p_ref[0, :N, :] + p_ref[1, :N, :]) * nd_ref[...] + b_ref[...]
    h = jnp.maximum(agg, 0.0)
    o_ref[pl.ds(0, N)] = (
        jnp.dot(h, w_ref[...], preferred_element_type=jnp.float32) * ns_ref[...]
    )
    o_ref[pl.ds(N, N_P - N)] = jnp.zeros((N_P - N, D), jnp.float32)


_mid_tc = pl.pallas_call(
    _mid_body,
    out_shape=jax.ShapeDtypeStruct((N_P, D), jnp.float32),
)


def _final_body(p_ref, nd_ref, b_ref, wfc_ref, bfc_ref, o_ref):
    agg = (p_ref[0, :N, :] + p_ref[1, :N, :]) * nd_ref[...] + b_ref[...]
    o_ref[...] = (
        jnp.dot(agg, wfc_ref[...], preferred_element_type=jnp.float32)
        + bfc_ref[...]
    )


_final_tc = pl.pallas_call(
    _final_body,
    out_shape=jax.ShapeDtypeStruct((N, NCLS), jnp.float32),
)


def kernel(inputs, edge_index, W1, b1, W2, b2, Wfc, bfc):
    pad = EPAD - EPW
    ei = edge_index.reshape(2, NW, EPW)
    pad_rows = jnp.broadcast_to(
        N + (jnp.arange(pad, dtype=jnp.int32) % (N_P - N)), (2, NW, pad))
    edges = jnp.concatenate([ei, pad_rows], axis=2).reshape(
        2, NW, NGRP, WIN, CHUNK)

    degp = _deg_kernel(edges)
    degp = degp.transpose(0, 2, 1, 3).reshape(NC, 2, N_P // 128, 128)
    norms = _norms_tc(degp)
    ns_col = norms[0].reshape(N_P, 1)[:N]
    nd_col = norms[1].reshape(N_P, 1)[:N]

    xh1 = _xw_scale_tc(inputs, W1, ns_col)
    p1 = _agg_kernel(xh1, edges)
    xh2 = _mid_tc(p1, nd_col, b1.reshape(1, D), W2, ns_col)
    p2 = _agg_kernel(xh2, edges)
    out = _final_tc(p2, nd_col, b2.reshape(1, D), Wfc, bfc.reshape(1, NCLS))
    return out
